# Initial kernel scaffold; baseline (speedup 1.0000x reference)
import jax
import jax.numpy as jnp
from jax.experimental import pallas as pl


def kernel(features_v1, ADJ_TOPO, z_pre, params, edge_index):
    raise NotImplementedError("write your pallas kernel here")


# speedup vs baseline: 4.7427x; 4.7427x over previous
import functools

import jax
import jax.numpy as jnp
from jax import lax
from jax.experimental import pallas as pl
from jax.experimental.pallas import tpu as pltpu
from jax.experimental.pallas import tpu_sc as plsc

N = 10000
E = 320000
D_IN = 128
H = 64
RANK = 32

NC = 2
NS = 16
NW = NC * NS

NPAD = 10240
ROWS_PT = NPAD // NS

CH = 128
NCH = 80
EPT = NCH * CH
EPAD = NW * EPT
IDX_ROWS = EPAD // CH

_mesh = plsc.VectorSubcoreMesh(
    core_axis_name="c", subcore_axis_name="s", num_cores=NC, num_subcores=NS)
_sc_params = pltpu.CompilerParams(use_tc_tiling_on_sc=False)


@functools.partial(
    pl.kernel,
    out_type=[
        jax.ShapeDtypeStruct((NC, NPAD), jnp.float32),
        jax.ShapeDtypeStruct((NC, NPAD), jnp.float32),
    ],
    mesh=_mesh,
    scratch_types=[
        pltpu.VMEM((NCH, CH), jnp.int32),
        pltpu.VMEM((NCH, CH), jnp.int32),
        pltpu.VMEM((CH,), jnp.float32),
        pltpu.VMEM_SHARED((NPAD,), jnp.float32),
        pltpu.VMEM_SHARED((NPAD,), jnp.float32),
    ],
    compiler_params=_sc_params,
)
def _sc_degrees(src_hbm, dst_hbm, ones_hbm, zrow_hbm,
                dego_hbm, degi_hbm,
                src_v, dst_v, ones_v, dego_sh, degi_sh):
    cid = lax.axis_index("c")
    sid = lax.axis_index("s")
    wid = sid * NC + cid
    sbase = sid * ROWS_PT

    pltpu.sync_copy(src_hbm.at[pl.ds(wid * NCH, NCH)], src_v)
    pltpu.sync_copy(dst_hbm.at[pl.ds(wid * NCH, NCH)], dst_v)
    pltpu.sync_copy(ones_hbm, ones_v)
    pltpu.sync_copy(zrow_hbm, dego_sh.at[pl.ds(sbase, ROWS_PT)])
    pltpu.sync_copy(zrow_hbm, degi_sh.at[pl.ds(sbase, ROWS_PT)])
    plsc.subcore_barrier()

    @pl.loop(0, NCH)
    def _chunks(j):
        pltpu.sync_copy(ones_v, dego_sh.at[src_v.at[j]], add=True)
        pltpu.sync_copy(ones_v, degi_sh.at[dst_v.at[j]], add=True)

    plsc.subcore_barrier()
    pltpu.sync_copy(dego_sh.at[pl.ds(sbase, ROWS_PT)],
                    dego_hbm.at[cid, pl.ds(sbase, ROWS_PT)])
    pltpu.sync_copy(degi_sh.at[pl.ds(sbase, ROWS_PT)],
                    degi_hbm.at[cid, pl.ds(sbase, ROWS_PT)])


@functools.partial(
    pl.kernel,
    out_type=jax.ShapeDtypeStruct((NC, NPAD, H), jnp.float32),
    mesh=_mesh,
    scratch_types=[
        pltpu.VMEM((NCH, CH), jnp.int32),
        pltpu.VMEM((NCH, CH), jnp.int32),
        pltpu.VMEM((2, CH, H), jnp.float32),
        pltpu.VMEM_SHARED((NPAD, H), jnp.float32),
        pltpu.SemaphoreType.DMA((2,)),
    ],
    compiler_params=_sc_params,
)
def _sc_scatter(xs_hbm, src_hbm, dst_hbm, zblk_hbm,
                y_hbm,
                src_v, dst_v, rows_v, y_sh, sem):
    cid = lax.axis_index("c")
    sid = lax.axis_index("s")
    wid = sid * NC + cid
    sbase = sid * ROWS_PT

    pltpu.sync_copy(src_hbm.at[pl.ds(wid * NCH, NCH)], src_v)
    pltpu.sync_copy(dst_hbm.at[pl.ds(wid * NCH, NCH)], dst_v)
    pltpu.sync_copy(zblk_hbm, y_sh.at[pl.ds(sbase, ROWS_PT)])
    plsc.subcore_barrier()

    pltpu.async_copy(xs_hbm.at[src_v.at[0]], rows_v.at[0], sem.at[0])

    @pl.loop(0, NCH, step=2)
    def _pairs(t):
        for b in range(2):
            j = t + b

            @pl.when(j < NCH)
            def _():
                pltpu.make_async_copy(
                    xs_hbm.at[pl.ds(0, CH)], rows_v.at[b], sem.at[b]).wait()

                @pl.when(j + 1 < NCH)
                def _():
                    pltpu.async_copy(
                        xs_hbm.at[src_v.at[j + 1]], rows_v.at[1 - b],
                        sem.at[1 - b])

                pltpu.sync_copy(rows_v.at[b], y_sh.at[dst_v.at[j]], add=True)

    plsc.subcore_barrier()
    pltpu.sync_copy(y_sh.at[pl.ds(sbase, ROWS_PT)],
                    y_hbm.at[cid, pl.ds(sbase, ROWS_PT)])


def _norm_cols(dego, degi):
    do = dego[0] + dego[1]
    di = degi[0] + degi[1]
    ns = jnp.where(do > 0.0, lax.rsqrt(do), 0.0)[:N, None]
    nd = jnp.where(di > 0.0, lax.rsqrt(di), 0.0)[:N, None]
    return ns, nd


def _gate(tx, wpt, bp, g):
    h = jnp.tanh(jnp.dot(tx, wpt, preferred_element_type=jnp.float32) + bp)
    return jnp.dot(h, g, preferred_element_type=jnp.float32) * (1.0 / RANK)


def _write_xs(xs_ref, tx, ns):
    xs_ref[:N, :] = tx * ns
    xs_ref[N:, :] = jnp.zeros((NPAD - N, H), jnp.float32)


def _t0_body(x_ref, wint_ref, bin_ref, wp0t_ref, bp0_ref, g0_ref,
             dego_ref, degi_ref,
             tx0_ref, xs0_ref, ns_ref, nd_ref, hid_ref):
    tx0 = jnp.dot(x_ref[...], wint_ref[...],
                  preferred_element_type=jnp.float32) + bin_ref[...]
    ns, nd = _norm_cols(dego_ref[...], degi_ref[...])
    eta0 = _gate(tx0, wp0t_ref[...], bp0_ref[...], g0_ref[...])
    tx0_ref[...] = tx0
    ns_ref[...] = ns
    nd_ref[...] = nd
    hid_ref[...] = tx0 * eta0
    _write_xs(xs0_ref, tx0, ns)


def _t1_body(y_ref, ns_ref, nd_ref, hid_in_ref, wct_ref, bc_ref,
             wpt_ref, bp_ref, g_ref,
             tx1_ref, xs1_ref, hid_ref):
    m = (y_ref[0, :N, :] + y_ref[1, :N, :]) * nd_ref[...]
    tx1 = jnp.dot(m, wct_ref[...],
                  preferred_element_type=jnp.float32) + bc_ref[...]
    eta = _gate(tx1, wpt_ref[...], bp_ref[...], g_ref[...])
    tx1_ref[...] = tx1
    hid_ref[...] = hid_in_ref[...] + tx1 * eta
    _write_xs(xs1_ref, tx1, ns_ref[...])


def _t2_body(y_ref, ns_ref, nd_ref, txprev_ref, hid_in_ref, wct_ref, bc_ref,
             wpt_ref, bp_ref, g_ref,
             tx2_ref, xs2_ref, hid_ref):
    m = (y_ref[0, :N, :] + y_ref[1, :N, :]) * nd_ref[...]
    c = jnp.dot(m, wct_ref[...],
                preferred_element_type=jnp.float32) + bc_ref[...]
    tx2 = 2.0 * c - txprev_ref[...]
    eta = _gate(tx2, wpt_ref[...], bp_ref[...], g_ref[...])
    tx2_ref[...] = tx2
    hid_ref[...] = hid_in_ref[...] + tx2 * eta
    _write_xs(xs2_ref, tx2, ns_ref[...])


def _t3_body(y_ref, nd_ref, txprev_ref, hid_in_ref, wct_ref, bc_ref,
             wpt_ref, bp_ref, g_ref,
             hid_ref):
    m = (y_ref[0, :N, :] + y_ref[1, :N, :]) * nd_ref[...]
    c = jnp.dot(m, wct_ref[...],
                preferred_element_type=jnp.float32) + bc_ref[...]
    tx3 = 2.0 * c - txprev_ref[...]
    eta = _gate(tx3, wpt_ref[...], bp_ref[...], g_ref[...])
    hid_ref[...] = hid_in_ref[...] + tx3 * eta


_f32 = jnp.float32

_t0_call = pl.pallas_call(
    _t0_body,
    out_shape=[
        jax.ShapeDtypeStruct((N, H), _f32),
        jax.ShapeDtypeStruct((NPAD, H), _f32),
        jax.ShapeDtypeStruct((N, 1), _f32),
        jax.ShapeDtypeStruct((N, 1), _f32),
        jax.ShapeDtypeStruct((N, H), _f32),
    ],
)

_t1_call = pl.pallas_call(
    _t1_body,
    out_shape=[
        jax.ShapeDtypeStruct((N, H), _f32),
        jax.ShapeDtypeStruct((NPAD, H), _f32),
        jax.ShapeDtypeStruct((N, H), _f32),
    ],
)

_t2_call = pl.pallas_call(
    _t2_body,
    out_shape=[
        jax.ShapeDtypeStruct((N, H), _f32),
        jax.ShapeDtypeStruct((NPAD, H), _f32),
        jax.ShapeDtypeStruct((N, H), _f32),
    ],
)

_t3_call = pl.pallas_call(
    _t3_body,
    out_shape=jax.ShapeDtypeStruct((N, H), _f32),
)


def kernel(features_v1, ADJ_TOPO, z_pre, params, edge_index):
    del ADJ_TOPO, z_pre
    p = params

    pad = jnp.full((EPAD - E,), N, jnp.int32)
    srcR = jnp.concatenate([edge_index[0], pad]).reshape(IDX_ROWS, CH)
    dstR = jnp.concatenate([edge_index[1], pad]).reshape(IDX_ROWS, CH)
    ones_row = jnp.ones((CH,), _f32)
    zrow = jnp.zeros((ROWS_PT,), _f32)
    zblk = jnp.zeros((ROWS_PT, H), _f32)

    wint = p['W_in'].T
    bin_ = p['b_in'][None, :]
    wc1t = p['Wc1'].T
    bc1 = p['bc1'][None, :]
    wc2t = p['Wc2'].T
    bc2 = p['bc2'][None, :]
    wpt = [p['Wp'][k].T for k in range(4)]
    bp = [p['bp'][k][None, :] for k in range(4)]
    g = [p['gamma'][:, k:k + 1] for k in range(4)]

    dego, degi = _sc_degrees(srcR, dstR, ones_row, zrow)

    tx0, xs0, ns, nd, hid = _t0_call(
        features_v1, wint, bin_, wpt[0], bp[0], g[0], dego, degi)

    y1 = _sc_scatter(xs0, srcR, dstR, zblk)
    tx1, xs1, hid = _t1_call(y1, ns, nd, hid, wc1t, bc1, wpt[1], bp[1], g[1])

    y2 = _sc_scatter(xs1, srcR, dstR, zblk)
    tx2, xs2, hid = _t2_call(y2, ns, nd, tx0, hid, wc2t, bc2,
                             wpt[2], bp[2], g[2])

    y3 = _sc_scatter(xs2, srcR, dstR, zblk)
    hid = _t3_call(y3, nd, tx1, hid, wc2t, bc2, wpt[3], bp[3], g[3])

    return hid


# speedup vs baseline: 11.5363x; 2.4325x over previous
import functools

import jax
import jax.numpy as jnp
from jax import lax
from jax.experimental import pallas as pl
from jax.experimental.pallas import tpu as pltpu
from jax.experimental.pallas import tpu_sc as plsc

N = 10000
E = 320000
D_IN = 128
H = 64
RANK = 32

NC = 2
NS = 16
NW = NC * NS

NPAD = 10240
ROWS_PT = NPAD // NS

CH = 128
NCH = 80
EPT = NCH * CH
EPAD = NW * EPT
IDX_ROWS = EPAD // CH

_mesh = plsc.VectorSubcoreMesh(
    core_axis_name="c", subcore_axis_name="s", num_cores=NC, num_subcores=NS)
_sc_params = pltpu.CompilerParams(use_tc_tiling_on_sc=False)


@functools.partial(
    pl.kernel,
    out_type=[
        jax.ShapeDtypeStruct((NC, NPAD), jnp.float32),
        jax.ShapeDtypeStruct((NC, NPAD), jnp.float32),
    ],
    mesh=_mesh,
    scratch_types=[
        pltpu.VMEM((NCH, CH), jnp.int32),
        pltpu.VMEM((NCH, CH), jnp.int32),
        pltpu.VMEM((CH,), jnp.float32),
        pltpu.VMEM_SHARED((NPAD,), jnp.float32),
        pltpu.VMEM_SHARED((NPAD,), jnp.float32),
    ],
    compiler_params=_sc_params,
)
def _sc_degrees(src_hbm, dst_hbm, ones_hbm, zrow_hbm,
                dego_hbm, degi_hbm,
                src_v, dst_v, ones_v, dego_sh, degi_sh):
    cid = lax.axis_index("c")
    sid = lax.axis_index("s")
    wid = sid * NC + cid
    sbase = sid * ROWS_PT

    pltpu.sync_copy(src_hbm.at[pl.ds(wid * NCH, NCH)], src_v)
    pltpu.sync_copy(dst_hbm.at[pl.ds(wid * NCH, NCH)], dst_v)
    pltpu.sync_copy(ones_hbm, ones_v)
    pltpu.sync_copy(zrow_hbm, dego_sh.at[pl.ds(sbase, ROWS_PT)])
    pltpu.sync_copy(zrow_hbm, degi_sh.at[pl.ds(sbase, ROWS_PT)])
    plsc.subcore_barrier()

    @pl.loop(0, NCH)
    def _chunks(j):
        pltpu.sync_copy(ones_v, dego_sh.at[src_v.at[j]], add=True)
        pltpu.sync_copy(ones_v, degi_sh.at[dst_v.at[j]], add=True)

    plsc.subcore_barrier()
    pltpu.sync_copy(dego_sh.at[pl.ds(sbase, ROWS_PT)],
                    dego_hbm.at[cid, pl.ds(sbase, ROWS_PT)])
    pltpu.sync_copy(degi_sh.at[pl.ds(sbase, ROWS_PT)],
                    degi_hbm.at[cid, pl.ds(sbase, ROWS_PT)])


@functools.partial(
    pl.kernel,
    out_type=jax.ShapeDtypeStruct((NC, NPAD, H), jnp.float32),
    mesh=_mesh,
    scratch_types=[
        pltpu.VMEM((NCH, CH), jnp.int32),
        pltpu.VMEM((NCH, CH), jnp.int32),
        pltpu.VMEM((2, CH, H), jnp.float32),
        pltpu.VMEM((CH, H), jnp.float32),
        pltpu.VMEM_SHARED((NPAD, H), jnp.float32),
        pltpu.VMEM_SHARED((NPAD, H), jnp.float32),
        pltpu.SemaphoreType.DMA((2,)),
        pltpu.SemaphoreType.DMA,
    ],
    compiler_params=_sc_params,
)
def _sc_scatter(xs_hbm, src_hbm, dst_hbm,
                y_hbm,
                src_v, dst_v, rows_v, zblk_v, xs_sh, y_sh, sem, ssem):
    cid = lax.axis_index("c")
    sid = lax.axis_index("s")
    wid = sid * NC + cid
    sbase = sid * ROWS_PT

    stage = pltpu.async_copy(
        xs_hbm.at[pl.ds(sbase, ROWS_PT)], xs_sh.at[pl.ds(sbase, ROWS_PT)],
        ssem)
    pltpu.sync_copy(src_hbm.at[pl.ds(wid * NCH, NCH)], src_v)
    pltpu.sync_copy(dst_hbm.at[pl.ds(wid * NCH, NCH)], dst_v)

    z16 = jnp.zeros((16,), jnp.float32)

    @pl.loop(0, CH)
    def _zrow(i):
        for c in range(H // 16):
            zblk_v[i, pl.ds(c * 16, 16)] = z16

    for t in range(ROWS_PT // CH):
        pltpu.sync_copy(zblk_v, y_sh.at[pl.ds(sbase + t * CH, CH)])
    stage.wait()
    plsc.subcore_barrier()

    pltpu.async_copy(xs_sh.at[src_v.at[0]], rows_v.at[0], sem.at[0])

    @pl.loop(0, NCH, step=2)
    def _pairs(t):
        for b in range(2):
            j = t + b

            @pl.when(j < NCH)
            def _():
                pltpu.make_async_copy(
                    xs_sh.at[pl.ds(0, CH)], rows_v.at[b], sem.at[b]).wait()

                @pl.when(j + 1 < NCH)
                def _():
                    pltpu.async_copy(
                        xs_sh.at[src_v.at[j + 1]], rows_v.at[1 - b],
                        sem.at[1 - b])

                pltpu.sync_copy(rows_v.at[b], y_sh.at[dst_v.at[j]], add=True)

    plsc.subcore_barrier()
    pltpu.sync_copy(y_sh.at[pl.ds(sbase, ROWS_PT)],
                    y_hbm.at[cid, pl.ds(sbase, ROWS_PT)])


def _norm_cols(dego, degi):
    do = dego[0] + dego[1]
    di = degi[0] + degi[1]
    ns = jnp.where(do > 0.0, lax.rsqrt(do), 0.0)[:N, None]
    nd = jnp.where(di > 0.0, lax.rsqrt(di), 0.0)[:N, None]
    return ns, nd


def _gate(tx, wpt, bp, g):
    h = jnp.tanh(jnp.dot(tx, wpt, preferred_element_type=jnp.float32) + bp)
    return jnp.dot(h, g, preferred_element_type=jnp.float32) * (1.0 / RANK)


def _write_xs(xs_ref, tx, ns):
    xs_ref[:N, :] = tx * ns
    xs_ref[N:, :] = jnp.zeros((NPAD - N, H), jnp.float32)


def _t0_body(x_ref, wint_ref, bin_ref, wp0t_ref, bp0_ref, g0_ref,
             dego_ref, degi_ref,
             tx0_ref, xs0_ref, ns_ref, nd_ref, hid_ref):
    tx0 = jnp.dot(x_ref[...], wint_ref[...],
                  preferred_element_type=jnp.float32) + bin_ref[...]
    ns, nd = _norm_cols(dego_ref[...], degi_ref[...])
    eta0 = _gate(tx0, wp0t_ref[...], bp0_ref[...], g0_ref[...])
    tx0_ref[...] = tx0
    ns_ref[...] = ns
    nd_ref[...] = nd
    hid_ref[...] = tx0 * eta0
    _write_xs(xs0_ref, tx0, ns)


def _t1_body(y_ref, ns_ref, nd_ref, hid_in_ref, wct_ref, bc_ref,
             wpt_ref, bp_ref, g_ref,
             tx1_ref, xs1_ref, hid_ref):
    m = (y_ref[0, :N, :] + y_ref[1, :N, :]) * nd_ref[...]
    tx1 = jnp.dot(m, wct_ref[...],
                  preferred_element_type=jnp.float32) + bc_ref[...]
    eta = _gate(tx1, wpt_ref[...], bp_ref[...], g_ref[...])
    tx1_ref[...] = tx1
    hid_ref[...] = hid_in_ref[...] + tx1 * eta
    _write_xs(xs1_ref, tx1, ns_ref[...])


def _t2_body(y_ref, ns_ref, nd_ref, txprev_ref, hid_in_ref, wct_ref, bc_ref,
             wpt_ref, bp_ref, g_ref,
             tx2_ref, xs2_ref, hid_ref):
    m = (y_ref[0, :N, :] + y_ref[1, :N, :]) * nd_ref[...]
    c = jnp.dot(m, wct_ref[...],
                preferred_element_type=jnp.float32) + bc_ref[...]
    tx2 = 2.0 * c - txprev_ref[...]
    eta = _gate(tx2, wpt_ref[...], bp_ref[...], g_ref[...])
    tx2_ref[...] = tx2
    hid_ref[...] = hid_in_ref[...] + tx2 * eta
    _write_xs(xs2_ref, tx2, ns_ref[...])


def _t3_body(y_ref, nd_ref, txprev_ref, hid_in_ref, wct_ref, bc_ref,
             wpt_ref, bp_ref, g_ref,
             hid_ref):
    m = (y_ref[0, :N, :] + y_ref[1, :N, :]) * nd_ref[...]
    c = jnp.dot(m, wct_ref[...],
                preferred_element_type=jnp.float32) + bc_ref[...]
    tx3 = 2.0 * c - txprev_ref[...]
    eta = _gate(tx3, wpt_ref[...], bp_ref[...], g_ref[...])
    hid_ref[...] = hid_in_ref[...] + tx3 * eta


_f32 = jnp.float32

_t0_call = pl.pallas_call(
    _t0_body,
    out_shape=[
        jax.ShapeDtypeStruct((N, H), _f32),
        jax.ShapeDtypeStruct((NPAD, H), _f32),
        jax.ShapeDtypeStruct((N, 1), _f32),
        jax.ShapeDtypeStruct((N, 1), _f32),
        jax.ShapeDtypeStruct((N, H), _f32),
    ],
)

_t1_call = pl.pallas_call(
    _t1_body,
    out_shape=[
        jax.ShapeDtypeStruct((N, H), _f32),
        jax.ShapeDtypeStruct((NPAD, H), _f32),
        jax.ShapeDtypeStruct((N, H), _f32),
    ],
)

_t2_call = pl.pallas_call(
    _t2_body,
    out_shape=[
        jax.ShapeDtypeStruct((N, H), _f32),
        jax.ShapeDtypeStruct((NPAD, H), _f32),
        jax.ShapeDtypeStruct((N, H), _f32),
    ],
)

_t3_call = pl.pallas_call(
    _t3_body,
    out_shape=jax.ShapeDtypeStruct((N, H), _f32),
)


def kernel(features_v1, ADJ_TOPO, z_pre, params, edge_index):
    del ADJ_TOPO, z_pre
    p = params

    pad = jnp.full((EPAD - E,), N, jnp.int32)
    srcR = jnp.concatenate([edge_index[0], pad]).reshape(IDX_ROWS, CH)
    dstR = jnp.concatenate([edge_index[1], pad]).reshape(IDX_ROWS, CH)
    ones_row = jnp.ones((CH,), _f32)
    zrow = jnp.zeros((ROWS_PT,), _f32)

    wint = p['W_in'].T
    bin_ = p['b_in'][None, :]
    wc1t = p['Wc1'].T
    bc1 = p['bc1'][None, :]
    wc2t = p['Wc2'].T
    bc2 = p['bc2'][None, :]
    wpt = [p['Wp'][k].T for k in range(4)]
    bp = [p['bp'][k][None, :] for k in range(4)]
    g = [p['gamma'][:, k:k + 1] for k in range(4)]

    dego, degi = _sc_degrees(srcR, dstR, ones_row, zrow)

    tx0, xs0, ns, nd, hid = _t0_call(
        features_v1, wint, bin_, wpt[0], bp[0], g[0], dego, degi)

    y1 = _sc_scatter(xs0, srcR, dstR)
    tx1, xs1, hid = _t1_call(y1, ns, nd, hid, wc1t, bc1, wpt[1], bp[1], g[1])

    y2 = _sc_scatter(xs1, srcR, dstR)
    tx2, xs2, hid = _t2_call(y2, ns, nd, tx0, hid, wc2t, bc2,
                             wpt[2], bp[2], g[2])

    y3 = _sc_scatter(xs2, srcR, dstR)
    hid = _t3_call(y3, nd, tx1, hid, wc2t, bc2, wpt[3], bp[3], g[3])

    return hid


# speedup vs baseline: 12.8448x; 1.1134x over previous
import functools

import jax
import jax.numpy as jnp
from jax import lax
from jax.experimental import pallas as pl
from jax.experimental.pallas import tpu as pltpu
from jax.experimental.pallas import tpu_sc as plsc

N = 10000
E = 320000
D_IN = 128
H = 64
RANK = 32

NC = 2
NS = 16
NW = NC * NS

NPAD = 10240
ROWS_PT = NPAD // NS

CH = 125
NCHD = 80
NCHS = 160
IDX_ROWS = E // CH
HW = H // 2
NBUF = 5
PRE = 3
ZR = 40

_mesh = plsc.VectorSubcoreMesh(
    core_axis_name="c", subcore_axis_name="s", num_cores=NC, num_subcores=NS)
_sc_params = pltpu.CompilerParams(use_tc_tiling_on_sc=False)


@functools.partial(
    pl.kernel,
    out_type=[
        jax.ShapeDtypeStruct((NC, NPAD), jnp.float32),
        jax.ShapeDtypeStruct((NC, NPAD), jnp.float32),
    ],
    mesh=_mesh,
    scratch_types=[
        pltpu.VMEM((NCHD, CH), jnp.int32),
        pltpu.VMEM((NCHD, CH), jnp.int32),
        pltpu.VMEM((CH,), jnp.float32),
        pltpu.VMEM_SHARED((NPAD,), jnp.float32),
        pltpu.VMEM_SHARED((NPAD,), jnp.float32),
    ],
    compiler_params=_sc_params,
)
def _sc_degrees(src_hbm, dst_hbm, ones_hbm, zrow_hbm,
                dego_hbm, degi_hbm,
                src_v, dst_v, ones_v, dego_sh, degi_sh):
    cid = lax.axis_index("c")
    sid = lax.axis_index("s")
    wid = sid * NC + cid
    sbase = sid * ROWS_PT

    pltpu.sync_copy(src_hbm.at[pl.ds(wid * NCHD, NCHD)], src_v)
    pltpu.sync_copy(dst_hbm.at[pl.ds(wid * NCHD, NCHD)], dst_v)
    pltpu.sync_copy(ones_hbm, ones_v)
    pltpu.sync_copy(zrow_hbm, dego_sh.at[pl.ds(sbase, ROWS_PT)])
    pltpu.sync_copy(zrow_hbm, degi_sh.at[pl.ds(sbase, ROWS_PT)])
    plsc.subcore_barrier()

    @pl.loop(0, NCHD)
    def _chunks(j):
        pltpu.sync_copy(ones_v, dego_sh.at[src_v.at[j]], add=True)
        pltpu.sync_copy(ones_v, degi_sh.at[dst_v.at[j]], add=True)

    plsc.subcore_barrier()
    pltpu.sync_copy(dego_sh.at[pl.ds(sbase, ROWS_PT)],
                    dego_hbm.at[cid, pl.ds(sbase, ROWS_PT)])
    pltpu.sync_copy(degi_sh.at[pl.ds(sbase, ROWS_PT)],
                    degi_hbm.at[cid, pl.ds(sbase, ROWS_PT)])


@functools.partial(
    pl.kernel,
    out_type=jax.ShapeDtypeStruct((NC, NPAD, HW), jnp.float32),
    mesh=_mesh,
    scratch_types=[
        pltpu.VMEM((NCHS, CH), jnp.int32),
        pltpu.VMEM((NCHS, CH), jnp.int32),
        pltpu.VMEM((NBUF, CH, HW), jnp.float32),
        pltpu.VMEM((ZR, HW), jnp.float32),
        pltpu.VMEM_SHARED((NPAD, HW), jnp.float32),
        pltpu.VMEM_SHARED((NPAD, HW), jnp.float32),
        pltpu.SemaphoreType.DMA((NBUF,)),
        pltpu.SemaphoreType.DMA((NBUF,)),
        pltpu.SemaphoreType.DMA,
    ],
    compiler_params=_sc_params,
)
def _sc_scatter(xs_hbm, src_hbm, dst_hbm,
                y_hbm,
                src_v, dst_v, rows_v, zblk_v, xs_sh, y_sh, gsem, ssem, psem):
    cid = lax.axis_index("c")
    sid = lax.axis_index("s")
    sbase = sid * ROWS_PT

    stage = pltpu.async_copy(
        xs_hbm.at[cid, pl.ds(sbase, ROWS_PT)],
        xs_sh.at[pl.ds(sbase, ROWS_PT)], psem)
    pltpu.sync_copy(src_hbm.at[pl.ds(sid * NCHS, NCHS)], src_v)
    pltpu.sync_copy(dst_hbm.at[pl.ds(sid * NCHS, NCHS)], dst_v)

    z16 = jnp.zeros((16,), jnp.float32)

    @pl.loop(0, ZR)
    def _zrow(i):
        for c in range(HW // 16):
            zblk_v[i, pl.ds(c * 16, 16)] = z16

    for t in range(ROWS_PT // ZR):
        pltpu.sync_copy(zblk_v, y_sh.at[pl.ds(sbase + t * ZR, ZR)])
    stage.wait()
    plsc.subcore_barrier()

    def _wait_gather(b):
        pltpu.make_async_copy(
            xs_sh.at[pl.ds(0, CH)], rows_v.at[b], gsem.at[b]).wait()

    def _wait_scatter(b):
        pltpu.make_async_copy(
            rows_v.at[b], y_sh.at[pl.ds(0, CH)], ssem.at[b]).wait()

    for b in range(PRE):
        pltpu.async_copy(xs_sh.at[src_v.at[b]], rows_v.at[b], gsem.at[b])

    @pl.loop(0, NCHS, step=NBUF)
    def _ring(t):
        for b in range(NBUF):
            j = t + b
            bn = (b + PRE) % NBUF

            @pl.when(j + PRE < NCHS)
            def _():
                @pl.when(j >= NBUF - PRE)
                def _():
                    _wait_scatter(bn)

                pltpu.async_copy(
                    xs_sh.at[src_v.at[j + PRE]], rows_v.at[bn],
                    gsem.at[bn])

            _wait_gather(b)
            pltpu.async_copy(rows_v.at[b], y_sh.at[dst_v.at[j]],
                             ssem.at[b], add=True)

    for b in range(NBUF):
        _wait_scatter(b)

    plsc.subcore_barrier()
    pltpu.sync_copy(y_sh.at[pl.ds(sbase, ROWS_PT)],
                    y_hbm.at[cid, pl.ds(sbase, ROWS_PT)])


def _gate(tx, wpt, bp, g):
    h = jnp.tanh(jnp.dot(tx, wpt, preferred_element_type=jnp.float32) + bp)
    return jnp.dot(h, g, preferred_element_type=jnp.float32) * (1.0 / RANK)


def _write_xs(xs_ref, tx, ns):
    xsv = tx * ns
    xs_ref[0, :N, :] = xsv[:, :HW]
    xs_ref[1, :N, :] = xsv[:, HW:]
    xs_ref[0, N:, :] = jnp.zeros((NPAD - N, HW), jnp.float32)
    xs_ref[1, N:, :] = jnp.zeros((NPAD - N, HW), jnp.float32)


def _ysum(y_ref, nd):
    return jnp.concatenate([y_ref[0, :N, :], y_ref[1, :N, :]], axis=1) * nd


def _t0a_body(x_ref, wint_ref, bin_ref, wp0t_ref, bp0_ref, g0_ref,
              tx0_ref, hid_ref):
    tx0 = jnp.dot(x_ref[...], wint_ref[...],
                  preferred_element_type=jnp.float32) + bin_ref[...]
    eta0 = _gate(tx0, wp0t_ref[...], bp0_ref[...], g0_ref[...])
    tx0_ref[...] = tx0
    hid_ref[...] = tx0 * eta0


def _t0b_body(tx0_ref, dego_ref, degi_ref,
              xs0_ref, ns_ref, nd_ref):
    do = dego_ref[0] + dego_ref[1]
    di = degi_ref[0] + degi_ref[1]
    ns = jnp.where(do > 0.0, lax.rsqrt(do), 0.0)[:N, None]
    nd = jnp.where(di > 0.0, lax.rsqrt(di), 0.0)[:N, None]
    ns_ref[...] = ns
    nd_ref[...] = nd
    _write_xs(xs0_ref, tx0_ref[...], ns)


def _t1_body(y_ref, ns_ref, nd_ref, hid_in_ref, wct_ref, bc_ref,
             wpt_ref, bp_ref, g_ref,
             tx1_ref, xs1_ref, hid_ref):
    m = _ysum(y_ref, nd_ref[...])
    tx1 = jnp.dot(m, wct_ref[...],
                  preferred_element_type=jnp.float32) + bc_ref[...]
    eta = _gate(tx1, wpt_ref[...], bp_ref[...], g_ref[...])
    tx1_ref[...] = tx1
    hid_ref[...] = hid_in_ref[...] + tx1 * eta
    _write_xs(xs1_ref, tx1, ns_ref[...])


def _t2_body(y_ref, ns_ref, nd_ref, txprev_ref, hid_in_ref, wct_ref, bc_ref,
             wpt_ref, bp_ref, g_ref,
             tx2_ref, xs2_ref, hid_ref):
    m = _ysum(y_ref, nd_ref[...])
    c = jnp.dot(m, wct_ref[...],
                preferred_element_type=jnp.float32) + bc_ref[...]
    tx2 = 2.0 * c - txprev_ref[...]
    eta = _gate(tx2, wpt_ref[...], bp_ref[...], g_ref[...])
    tx2_ref[...] = tx2
    hid_ref[...] = hid_in_ref[...] + tx2 * eta
    _write_xs(xs2_ref, tx2, ns_ref[...])


def _t3_body(y_ref, nd_ref, txprev_ref, hid_in_ref, wct_ref, bc_ref,
             wpt_ref, bp_ref, g_ref,
             hid_ref):
    m = _ysum(y_ref, nd_ref[...])
    c = jnp.dot(m, wct_ref[...],
                preferred_element_type=jnp.float32) + bc_ref[...]
    tx3 = 2.0 * c - txprev_ref[...]
    eta = _gate(tx3, wpt_ref[...], bp_ref[...], g_ref[...])
    hid_ref[...] = hid_in_ref[...] + tx3 * eta


_f32 = jnp.float32

_t0a_call = pl.pallas_call(
    _t0a_body,
    out_shape=[
        jax.ShapeDtypeStruct((N, H), _f32),
        jax.ShapeDtypeStruct((N, H), _f32),
    ],
)

_t0b_call = pl.pallas_call(
    _t0b_body,
    out_shape=[
        jax.ShapeDtypeStruct((NC, NPAD, HW), _f32),
        jax.ShapeDtypeStruct((N, 1), _f32),
        jax.ShapeDtypeStruct((N, 1), _f32),
    ],
)

_t1_call = pl.pallas_call(
    _t1_body,
    out_shape=[
        jax.ShapeDtypeStruct((N, H), _f32),
        jax.ShapeDtypeStruct((NC, NPAD, HW), _f32),
        jax.ShapeDtypeStruct((N, H), _f32),
    ],
)

_t2_call = pl.pallas_call(
    _t2_body,
    out_shape=[
        jax.ShapeDtypeStruct((N, H), _f32),
        jax.ShapeDtypeStruct((NC, NPAD, HW), _f32),
        jax.ShapeDtypeStruct((N, H), _f32),
    ],
)

_t3_call = pl.pallas_call(
    _t3_body,
    out_shape=jax.ShapeDtypeStruct((N, H), _f32),
)


def kernel(features_v1, ADJ_TOPO, z_pre, params, edge_index):
    del ADJ_TOPO, z_pre
    p = params

    srcR = edge_index[0].reshape(IDX_ROWS, CH)
    dstR = edge_index[1].reshape(IDX_ROWS, CH)
    ones_row = jnp.ones((CH,), _f32)
    zrow = jnp.zeros((ROWS_PT,), _f32)

    wint = p['W_in'].T
    bin_ = p['b_in'][None, :]
    wc1t = p['Wc1'].T
    bc1 = p['bc1'][None, :]
    wc2t = p['Wc2'].T
    bc2 = p['bc2'][None, :]
    wpt = [p['Wp'][k].T for k in range(4)]
    bp = [p['bp'][k][None, :] for k in range(4)]
    g = [p['gamma'][:, k:k + 1] for k in range(4)]

    dego, degi = _sc_degrees(srcR, dstR, ones_row, zrow)
    tx0, hid = _t0a_call(features_v1, wint, bin_, wpt[0], bp[0], g[0])
    xs0, ns, nd = _t0b_call(tx0, dego, degi)

    y1 = _sc_scatter(xs0, srcR, dstR)
    tx1, xs1, hid = _t1_call(y1, ns, nd, hid, wc1t, bc1, wpt[1], bp[1], g[1])

    y2 = _sc_scatter(xs1, srcR, dstR)
    tx2, xs2, hid = _t2_call(y2, ns, nd, tx0, hid, wc2t, bc2,
                             wpt[2], bp[2], g[2])

    y3 = _sc_scatter(xs2, srcR, dstR)
    hid = _t3_call(y3, nd, tx1, hid, wc2t, bc2, wpt[3], bp[3], g[3])

    return hid


# speedup vs baseline: 12.9243x; 1.0062x over previous
import functools

import jax
import jax.numpy as jnp
from jax import lax
from jax.experimental import pallas as pl
from jax.experimental.pallas import tpu as pltpu
from jax.experimental.pallas import tpu_sc as plsc

N = 10000
E = 320000
D_IN = 128
H = 64
RANK = 32

NC = 2
NS = 16
NW = NC * NS

NPAD = 10240
ROWS_PT = NPAD // NS

CH = 125
NCHD = 80
NCHS = 160
IDX_ROWS = E // CH
HW = H // 2
NBUF = 5
PRE = 3
ZR = 40

_mesh = plsc.VectorSubcoreMesh(
    core_axis_name="c", subcore_axis_name="s", num_cores=NC, num_subcores=NS)
_sc_params = pltpu.CompilerParams(use_tc_tiling_on_sc=False)


@functools.partial(
    pl.kernel,
    out_type=[
        jax.ShapeDtypeStruct((NC, NPAD), jnp.float32),
        jax.ShapeDtypeStruct((NC, NPAD), jnp.float32),
    ],
    mesh=_mesh,
    scratch_types=[
        pltpu.VMEM((NCHD, CH), jnp.int32),
        pltpu.VMEM((NCHD, CH), jnp.int32),
        pltpu.VMEM((CH,), jnp.float32),
        pltpu.VMEM_SHARED((NPAD,), jnp.float32),
        pltpu.VMEM_SHARED((NPAD,), jnp.float32),
    ],
    compiler_params=_sc_params,
)
def _sc_degrees(src_hbm, dst_hbm, ones_hbm, zrow_hbm,
                dego_hbm, degi_hbm,
                src_v, dst_v, ones_v, dego_sh, degi_sh):
    cid = lax.axis_index("c")
    sid = lax.axis_index("s")
    wid = sid * NC + cid
    sbase = sid * ROWS_PT

    pltpu.sync_copy(src_hbm.at[pl.ds(wid * NCHD, NCHD)], src_v)
    pltpu.sync_copy(dst_hbm.at[pl.ds(wid * NCHD, NCHD)], dst_v)
    pltpu.sync_copy(ones_hbm, ones_v)
    pltpu.sync_copy(zrow_hbm, dego_sh.at[pl.ds(sbase, ROWS_PT)])
    pltpu.sync_copy(zrow_hbm, degi_sh.at[pl.ds(sbase, ROWS_PT)])
    plsc.subcore_barrier()

    @pl.loop(0, NCHD)
    def _chunks(j):
        pltpu.sync_copy(ones_v, dego_sh.at[src_v.at[j]], add=True)
        pltpu.sync_copy(ones_v, degi_sh.at[dst_v.at[j]], add=True)

    plsc.subcore_barrier()
    pltpu.sync_copy(dego_sh.at[pl.ds(sbase, ROWS_PT)],
                    dego_hbm.at[cid, pl.ds(sbase, ROWS_PT)])
    pltpu.sync_copy(degi_sh.at[pl.ds(sbase, ROWS_PT)],
                    degi_hbm.at[cid, pl.ds(sbase, ROWS_PT)])


@functools.partial(
    pl.kernel,
    out_type=jax.ShapeDtypeStruct((NC, NPAD, HW), jnp.float32),
    mesh=_mesh,
    scratch_types=[
        pltpu.VMEM((NCHS, CH), jnp.int32),
        pltpu.VMEM((NCHS, CH), jnp.int32),
        pltpu.VMEM((NBUF, CH, HW), jnp.float32),
        pltpu.VMEM((ZR, HW), jnp.float32),
        pltpu.VMEM_SHARED((NPAD, HW), jnp.float32),
        pltpu.VMEM_SHARED((NPAD, HW), jnp.float32),
        pltpu.SemaphoreType.DMA((NBUF,)),
        pltpu.SemaphoreType.DMA((NBUF,)),
        pltpu.SemaphoreType.DMA,
    ],
    compiler_params=_sc_params,
)
def _sc_scatter(xs_hbm, src_hbm, dst_hbm,
                y_hbm,
                src_v, dst_v, rows_v, zblk_v, xs_sh, y_sh, gsem, ssem, psem):
    cid = lax.axis_index("c")
    sid = lax.axis_index("s")
    sbase = sid * ROWS_PT

    stage = pltpu.async_copy(
        xs_hbm.at[cid, pl.ds(sbase, ROWS_PT)],
        xs_sh.at[pl.ds(sbase, ROWS_PT)], psem)
    pltpu.sync_copy(src_hbm.at[pl.ds(sid * NCHS, NCHS)], src_v)
    pltpu.sync_copy(dst_hbm.at[pl.ds(sid * NCHS, NCHS)], dst_v)

    z16 = jnp.zeros((16,), jnp.float32)

    @pl.loop(0, ZR)
    def _zrow(i):
        for c in range(HW // 16):
            zblk_v[i, pl.ds(c * 16, 16)] = z16

    for t in range(ROWS_PT // ZR):
        pltpu.sync_copy(zblk_v, y_sh.at[pl.ds(sbase + t * ZR, ZR)])
    stage.wait()
    plsc.subcore_barrier()

    def _wait_gather(b):
        pltpu.make_async_copy(
            xs_sh.at[pl.ds(0, CH)], rows_v.at[b], gsem.at[b]).wait()

    def _wait_scatter(b):
        pltpu.make_async_copy(
            rows_v.at[b], y_sh.at[pl.ds(0, CH)], ssem.at[b]).wait()

    for b in range(PRE):
        pltpu.async_copy(xs_sh.at[src_v.at[b]], rows_v.at[b], gsem.at[b])

    @pl.loop(0, NCHS, step=NBUF)
    def _ring(t):
        for b in range(NBUF):
            j = t + b
            bn = (b + PRE) % NBUF

            @pl.when(j + PRE < NCHS)
            def _():
                @pl.when(j >= NBUF - PRE)
                def _():
                    _wait_scatter(bn)

                pltpu.async_copy(
                    xs_sh.at[src_v.at[j + PRE]], rows_v.at[bn],
                    gsem.at[bn])

            _wait_gather(b)
            pltpu.async_copy(rows_v.at[b], y_sh.at[dst_v.at[j]],
                             ssem.at[b], add=True)

    for b in range(NBUF):
        _wait_scatter(b)

    plsc.subcore_barrier()
    pltpu.sync_copy(y_sh.at[pl.ds(sbase, ROWS_PT)],
                    y_hbm.at[cid, pl.ds(sbase, ROWS_PT)])


def _gate(tx, wpt, bp, g):
    h = jnp.tanh(jnp.dot(tx, wpt, preferred_element_type=jnp.float32) + bp)
    return jnp.dot(h, g, preferred_element_type=jnp.float32) * (1.0 / RANK)


def _write_xs(xs_ref, tx, ns):
    xsv = tx * ns
    xs_ref[0, :N, :] = xsv[:, :HW]
    xs_ref[1, :N, :] = xsv[:, HW:]
    xs_ref[0, N:, :] = jnp.zeros((NPAD - N, HW), jnp.float32)
    xs_ref[1, N:, :] = jnp.zeros((NPAD - N, HW), jnp.float32)


def _conv(y_ref, nd, wct):
    s = (jnp.dot(y_ref[0, :N, :], wct[:HW, :],
                 preferred_element_type=jnp.float32) +
         jnp.dot(y_ref[1, :N, :], wct[HW:, :],
                 preferred_element_type=jnp.float32))
    return s * nd


def _t0a_body(x_ref, wint_ref, bin_ref, wp0t_ref, bp0_ref, g0_ref,
              tx0_ref, hid_ref):
    tx0 = jnp.dot(x_ref[...], wint_ref[...],
                  preferred_element_type=jnp.float32) + bin_ref[...]
    eta0 = _gate(tx0, wp0t_ref[...], bp0_ref[...], g0_ref[...])
    tx0_ref[...] = tx0
    hid_ref[...] = tx0 * eta0


def _t0b_body(tx0_ref, dego_ref, degi_ref,
              xs0_ref, ns_ref, nd_ref):
    do = dego_ref[0] + dego_ref[1]
    di = degi_ref[0] + degi_ref[1]
    ns = jnp.where(do > 0.0, lax.rsqrt(do), 0.0)[:N, None]
    nd = jnp.where(di > 0.0, lax.rsqrt(di), 0.0)[:N, None]
    ns_ref[...] = ns
    nd_ref[...] = nd
    _write_xs(xs0_ref, tx0_ref[...], ns)


def _t1_body(y_ref, ns_ref, nd_ref, hid_in_ref, wct_ref, bc_ref,
             wpt_ref, bp_ref, g_ref,
             tx1_ref, xs1_ref, hid_ref):
    tx1 = _conv(y_ref, nd_ref[...], wct_ref) + bc_ref[...]
    eta = _gate(tx1, wpt_ref[...], bp_ref[...], g_ref[...])
    tx1_ref[...] = tx1
    hid_ref[...] = hid_in_ref[...] + tx1 * eta
    _write_xs(xs1_ref, tx1, ns_ref[...])


def _t2_body(y_ref, ns_ref, nd_ref, txprev_ref, hid_in_ref, wct_ref, bc_ref,
             wpt_ref, bp_ref, g_ref,
             tx2_ref, xs2_ref, hid_ref):
    c = _conv(y_ref, nd_ref[...], wct_ref) + bc_ref[...]
    tx2 = 2.0 * c - txprev_ref[...]
    eta = _gate(tx2, wpt_ref[...], bp_ref[...], g_ref[...])
    tx2_ref[...] = tx2
    hid_ref[...] = hid_in_ref[...] + tx2 * eta
    _write_xs(xs2_ref, tx2, ns_ref[...])


def _t3_body(y_ref, nd_ref, txprev_ref, hid_in_ref, wct_ref, bc_ref,
             wpt_ref, bp_ref, g_ref,
             hid_ref):
    c = _conv(y_ref, nd_ref[...], wct_ref) + bc_ref[...]
    tx3 = 2.0 * c - txprev_ref[...]
    eta = _gate(tx3, wpt_ref[...], bp_ref[...], g_ref[...])
    hid_ref[...] = hid_in_ref[...] + tx3 * eta


_f32 = jnp.float32

_t0a_call = pl.pallas_call(
    _t0a_body,
    out_shape=[
        jax.ShapeDtypeStruct((N, H), _f32),
        jax.ShapeDtypeStruct((N, H), _f32),
    ],
)

_t0b_call = pl.pallas_call(
    _t0b_body,
    out_shape=[
        jax.ShapeDtypeStruct((NC, NPAD, HW), _f32),
        jax.ShapeDtypeStruct((N, 1), _f32),
        jax.ShapeDtypeStruct((N, 1), _f32),
    ],
)

_tc_params = pltpu.CompilerParams(vmem_limit_bytes=100 * 1024 * 1024)

_t1_call = pl.pallas_call(
    _t1_body,
    compiler_params=_tc_params,
    out_shape=[
        jax.ShapeDtypeStruct((N, H), _f32),
        jax.ShapeDtypeStruct((NC, NPAD, HW), _f32),
        jax.ShapeDtypeStruct((N, H), _f32),
    ],
)

_t2_call = pl.pallas_call(
    _t2_body,
    compiler_params=_tc_params,
    out_shape=[
        jax.ShapeDtypeStruct((N, H), _f32),
        jax.ShapeDtypeStruct((NC, NPAD, HW), _f32),
        jax.ShapeDtypeStruct((N, H), _f32),
    ],
)

_t3_call = pl.pallas_call(
    _t3_body,
    compiler_params=_tc_params,
    out_shape=jax.ShapeDtypeStruct((N, H), _f32),
)


def kernel(features_v1, ADJ_TOPO, z_pre, params, edge_index):
    del ADJ_TOPO, z_pre
    p = params

    srcR = edge_index[0].reshape(IDX_ROWS, CH)
    dstR = edge_index[1].reshape(IDX_ROWS, CH)
    ones_row = jnp.ones((CH,), _f32)
    zrow = jnp.zeros((ROWS_PT,), _f32)

    wint = p['W_in'].T
    bin_ = p['b_in'][None, :]
    wc1t = p['Wc1'].T
    bc1 = p['bc1'][None, :]
    wc2t = p['Wc2'].T
    bc2 = p['bc2'][None, :]
    wpt = [p['Wp'][k].T for k in range(4)]
    bp = [p['bp'][k][None, :] for k in range(4)]
    g = [p['gamma'][:, k:k + 1] for k in range(4)]

    dego, degi = _sc_degrees(srcR, dstR, ones_row, zrow)
    tx0, hid = _t0a_call(features_v1, wint, bin_, wpt[0], bp[0], g[0])
    xs0, ns, nd = _t0b_call(tx0, dego, degi)

    y1 = _sc_scatter(xs0, srcR, dstR)
    tx1, xs1, hid = _t1_call(y1, ns, nd, hid, wc1t, bc1, wpt[1], bp[1], g[1])

    y2 = _sc_scatter(xs1, srcR, dstR)
    tx2, xs2, hid = _t2_call(y2, ns, nd, tx0, hid, wc2t, bc2,
                             wpt[2], bp[2], g[2])

    y3 = _sc_scatter(xs2, srcR, dstR)
    hid = _t3_call(y3, nd, tx1, hid, wc2t, bc2, wpt[3], bp[3], g[3])

    return hid
